# Initial kernel scaffold; baseline (speedup 1.0000x reference)
import functools

import jax
import jax.numpy as jnp
from jax import lax
from jax.experimental import pallas as pl
from jax.experimental.pallas import tpu as pltpu
from jax.experimental.pallas import tpu_sc as plsc

_N = 10000
_E = 320000
_H = 128
_DEPTH = 4
_MAX_IN_DEG = 10
_XP = 16
_C = 128
_NCHUNKS = _E // _C
_NC = 2
_NS = 16
_NW = _NC * _NS
_BE = 2000
_BN = 2000

_f32 = jnp.float32


def _silu(v):
    return v / (1.0 + jnp.exp(-v))


def _sc_gather(p, q, xpad, src, dst):
    mesh = plsc.VectorSubcoreMesh(core_axis_name="c", subcore_axis_name="s")
    out_type = (
        jax.ShapeDtypeStruct((_E, _H), _f32),
        jax.ShapeDtypeStruct((_E, _H), _f32),
        jax.ShapeDtypeStruct((_E, _XP), _f32),
        jax.ShapeDtypeStruct((_E, _XP), _f32),
    )
    scratch = [
        pltpu.VMEM((_C,), jnp.int32),
        pltpu.VMEM((_C,), jnp.int32),
        pltpu.VMEM((_C, _H), _f32),
        pltpu.VMEM((_C, _H), _f32),
        pltpu.VMEM((_C, _XP), _f32),
        pltpu.VMEM((_C, _XP), _f32),
        pltpu.SemaphoreType.DMA,
        pltpu.SemaphoreType.DMA,
        pltpu.SemaphoreType.DMA,
        pltpu.SemaphoreType.DMA,
    ]

    def body(p_h, q_h, x_h, src_h, dst_h, pg_h, qg_h, xs_h, xd_h,
             idx_s, idx_d, bufp, bufq, bufxs, bufxd, sp, sq, s1, s2):
        c = lax.axis_index("c")
        s = lax.axis_index("s")
        wid = s * _NC + c
        base_chunks = _NCHUNKS // _NW
        extra = _NCHUNKS - base_chunks * _NW
        nloc = base_chunks + jnp.where(wid < extra, 1, 0)

        def step(i, carry):
            base = (wid + _NW * i) * _C
            pltpu.sync_copy(dst_h.at[pl.ds(base, _C)], idx_d)
            pltpu.sync_copy(src_h.at[pl.ds(base, _C)], idx_s)
            cp = pltpu.async_copy(p_h.at[idx_d], bufp, sp)
            cq = pltpu.async_copy(q_h.at[idx_s], bufq, sq)
            cxs = pltpu.async_copy(x_h.at[idx_s], bufxs, s1)
            cxd = pltpu.async_copy(x_h.at[idx_d], bufxd, s2)
            cp.wait()
            cq.wait()
            cxs.wait()
            cxd.wait()
            pltpu.sync_copy(bufp, pg_h.at[pl.ds(base, _C)])
            pltpu.sync_copy(bufq, qg_h.at[pl.ds(base, _C)])
            pltpu.sync_copy(bufxs, xs_h.at[pl.ds(base, _C)])
            pltpu.sync_copy(bufxd, xd_h.at[pl.ds(base, _C)])
            return carry

        lax.fori_loop(0, nloc, step, 0)

    return pl.kernel(body, out_type=out_type, mesh=mesh, scratch_types=scratch)(
        p, q, xpad, src, dst)


def _sc_scatter(m, v, dst):
    mesh = plsc.VectorSubcoreMesh(core_axis_name="c", subcore_axis_name="s")
    out_type = (
        jax.ShapeDtypeStruct((_NC, _N, _H), _f32),
        jax.ShapeDtypeStruct((_NC, _N, _XP), _f32),
    )
    scratch = [
        pltpu.VMEM((_C,), jnp.int32),
        pltpu.VMEM((_C, _H), _f32),
        pltpu.VMEM((_C, _XP), _f32),
        pltpu.VMEM_SHARED((_N, _H), _f32),
        pltpu.VMEM_SHARED((_N, _XP), _f32),
    ]
    rpt = _N // _NS
    zc = 125

    def body(m_h, v_h, dst_h, aggm_h, aggx_h, idx_d, bufm, bufv, shm, shx):
        c = lax.axis_index("c")
        s = lax.axis_index("s")

        def zm(t, carry):
            r = t // (_H // 16)
            k = t % (_H // 16)
            bufm[r, pl.ds(k * 16, 16)] = jnp.zeros((16,), _f32)
            return carry

        lax.fori_loop(0, _C * (_H // 16), zm, 0)

        def zv(t, carry):
            bufv[t, :] = jnp.zeros((_XP,), _f32)
            return carry

        lax.fori_loop(0, _C, zv, 0)

        for r in range(rpt // zc):
            pltpu.sync_copy(bufm.at[pl.ds(0, zc)],
                            shm.at[pl.ds(s * rpt + r * zc, zc)])
            pltpu.sync_copy(bufv.at[pl.ds(0, zc)],
                            shx.at[pl.ds(s * rpt + r * zc, zc)])
        plsc.subcore_barrier()

        percore = _NCHUNKS // _NC
        base_t = percore // _NS
        extra = percore - base_t * _NS
        nloc = base_t + jnp.where(s < extra, 1, 0)

        def step(i, carry):
            base = (c + _NC * (s + _NS * i)) * _C
            pltpu.sync_copy(dst_h.at[pl.ds(base, _C)], idx_d)
            pltpu.sync_copy(m_h.at[pl.ds(base, _C)], bufm)
            pltpu.sync_copy(v_h.at[pl.ds(base, _C)], bufv)
            pltpu.sync_copy(bufm, shm.at[idx_d], add=True)
            pltpu.sync_copy(bufv, shx.at[idx_d], add=True)
            return carry

        lax.fori_loop(0, nloc, step, 0)
        plsc.subcore_barrier()

        pltpu.sync_copy(shm.at[pl.ds(s * rpt, rpt)],
                        aggm_h.at[c, pl.ds(s * rpt, rpt)])
        pltpu.sync_copy(shx.at[pl.ds(s * rpt, rpt)],
                        aggx_h.at[c, pl.ds(s * rpt, rpt)])

    return pl.kernel(body, out_type=out_type, mesh=mesh, scratch_types=scratch)(
        m, v, dst)


def _full2(shape):
    return pl.BlockSpec(shape, lambda i: (0, 0))


def _tc_embed(feat, Win, b_in, W1a, be1l, W1b):
    def body(f_r, win_r, bin_r, wa_r, ba_r, wb_r, h_r, p_r, q_r):
        h = jnp.dot(f_r[...], win_r[...], preferred_element_type=_f32) + bin_r[...]
        h_r[...] = h
        p_r[...] = jnp.dot(h, wa_r[...], preferred_element_type=_f32) + ba_r[...]
        q_r[...] = jnp.dot(h, wb_r[...], preferred_element_type=_f32)

    row = pl.BlockSpec((_BN, _H), lambda i: (i, 0))
    return pl.pallas_call(
        body,
        grid=(_N // _BN,),
        in_specs=[row, _full2((_H, _H)), _full2((1, _H)), _full2((_H, _H)),
                  _full2((1, _H)), _full2((_H, _H))],
        out_specs=[row, row, row],
        out_shape=[jax.ShapeDtypeStruct((_N, _H), _f32)] * 3,
    )(feat, Win, b_in.reshape(1, _H), W1a, be1l.reshape(1, _H), W1b)


def _tc_edge(pg, qg, xs, xd, w2row, We2l, be2l, Wc1l, bc1l, Wc2l, bc2l):
    def body(pg_r, qg_r, xs_r, xd_r, w2_r, we2_r, be2_r, wc1_r, bc1_r,
             wc2_r, bc2_r, m_r, v_r):
        diff = xd_r[...] - xs_r[...]
        r2 = jnp.sum(diff * diff, axis=-1, keepdims=True)
        e1 = pg_r[...] + qg_r[...] + r2 * w2_r[...]
        u = _silu(e1)
        m = _silu(jnp.dot(u, we2_r[...], preferred_element_type=_f32) + be2_r[...])
        t = _silu(jnp.dot(m, wc1_r[...], preferred_element_type=_f32) + bc1_r[...])
        cw = jnp.dot(t, wc2_r[...], preferred_element_type=_f32) + bc2_r[...]
        m_r[...] = m
        v_r[...] = diff * cw

    erow = pl.BlockSpec((_BE, _H), lambda i: (i, 0))
    xrow = pl.BlockSpec((_BE, _XP), lambda i: (i, 0))
    return pl.pallas_call(
        body,
        grid=(_E // _BE,),
        in_specs=[erow, erow, xrow, xrow, _full2((1, _H)), _full2((_H, _H)),
                  _full2((1, _H)), _full2((_H, _H)), _full2((1, _H)),
                  _full2((_H, 1)), _full2((1, 1))],
        out_specs=[erow, xrow],
        out_shape=[jax.ShapeDtypeStruct((_E, _H), _f32),
                   jax.ShapeDtypeStruct((_E, _XP), _f32)],
    )(pg, qg, xs, xd, w2row.reshape(1, _H), We2l, be2l.reshape(1, _H),
      Wc1l, bc1l.reshape(1, _H), Wc2l, bc2l.reshape(1, 1))


def _tc_node(h, x, aggm, aggx, Wn1a, Wn1b, bn1l, Wn2l, bn2l, Wa, ba, Wb):
    def body(h_r, x_r, am_r, ax_r, wn1a_r, wn1b_r, bn1_r, wn2_r, bn2_r,
             wa_r, ba_r, wb_r, h2_r, x2_r, p_r, q_r):
        am = am_r[0] + am_r[1]
        ax = ax_r[0] + ax_r[1]
        g = _silu(jnp.dot(h_r[...], wn1a_r[...], preferred_element_type=_f32)
                  + jnp.dot(am, wn1b_r[...], preferred_element_type=_f32)
                  + bn1_r[...])
        h2 = h_r[...] + jnp.dot(g, wn2_r[...], preferred_element_type=_f32) + bn2_r[...]
        h2_r[...] = h2
        x2_r[...] = x_r[...] + ax / _MAX_IN_DEG
        p_r[...] = jnp.dot(h2, wa_r[...], preferred_element_type=_f32) + ba_r[...]
        q_r[...] = jnp.dot(h2, wb_r[...], preferred_element_type=_f32)

    row = pl.BlockSpec((_BN, _H), lambda i: (i, 0))
    xrow = pl.BlockSpec((_BN, _XP), lambda i: (i, 0))
    amrow = pl.BlockSpec((_NC, _BN, _H), lambda i: (0, i, 0))
    axrow = pl.BlockSpec((_NC, _BN, _XP), lambda i: (0, i, 0))
    return pl.pallas_call(
        body,
        grid=(_N // _BN,),
        in_specs=[row, xrow, amrow, axrow, _full2((_H, _H)), _full2((_H, _H)),
                  _full2((1, _H)), _full2((_H, _H)), _full2((1, _H)),
                  _full2((_H, _H)), _full2((1, _H)), _full2((_H, _H))],
        out_specs=[row, xrow, row, row],
        out_shape=[jax.ShapeDtypeStruct((_N, _H), _f32),
                   jax.ShapeDtypeStruct((_N, _XP), _f32),
                   jax.ShapeDtypeStruct((_N, _H), _f32),
                   jax.ShapeDtypeStruct((_N, _H), _f32)],
    )(h, x, aggm, aggx, Wn1a, Wn1b, bn1l.reshape(1, _H), Wn2l,
      bn2l.reshape(1, _H), Wa, ba.reshape(1, _H), Wb)


def _tc_node_last(h, x, aggm, aggx, Wn1a, Wn1b, bn1l, Wn2l, bn2l, Wout, b_out):
    def body(h_r, x_r, am_r, ax_r, wn1a_r, wn1b_r, bn1_r, wn2_r, bn2_r,
             wo_r, bo_r, o_r, x2_r):
        am = am_r[0] + am_r[1]
        ax = ax_r[0] + ax_r[1]
        g = _silu(jnp.dot(h_r[...], wn1a_r[...], preferred_element_type=_f32)
                  + jnp.dot(am, wn1b_r[...], preferred_element_type=_f32)
                  + bn1_r[...])
        h2 = h_r[...] + jnp.dot(g, wn2_r[...], preferred_element_type=_f32) + bn2_r[...]
        o_r[...] = jnp.dot(h2, wo_r[...], preferred_element_type=_f32) + bo_r[...]
        x2_r[...] = x_r[...] + ax / _MAX_IN_DEG

    row = pl.BlockSpec((_BN, _H), lambda i: (i, 0))
    xrow = pl.BlockSpec((_BN, _XP), lambda i: (i, 0))
    amrow = pl.BlockSpec((_NC, _BN, _H), lambda i: (0, i, 0))
    axrow = pl.BlockSpec((_NC, _BN, _XP), lambda i: (0, i, 0))
    return pl.pallas_call(
        body,
        grid=(_N // _BN,),
        in_specs=[row, xrow, amrow, axrow, _full2((_H, _H)), _full2((_H, _H)),
                  _full2((1, _H)), _full2((_H, _H)), _full2((1, _H)),
                  _full2((_H, _H)), _full2((1, _H))],
        out_specs=[row, xrow],
        out_shape=[jax.ShapeDtypeStruct((_N, _H), _f32),
                   jax.ShapeDtypeStruct((_N, _XP), _f32)],
    )(h, x, aggm, aggx, Wn1a, Wn1b, bn1l.reshape(1, _H), Wn2l,
      bn2l.reshape(1, _H), Wout, b_out.reshape(1, _H))


def kernel(feat, coordinate, edge_index, Win, b_in, Wout, b_out,
           We1, be1, We2, be2, Wc1, bc1, Wc2, bc2, Wn1, bn1, Wn2, bn2):
    src = edge_index[0]
    dst = edge_index[1]
    x = jnp.pad(coordinate, ((0, 0), (0, _XP - 3)))

    h, p, q = _tc_embed(feat, Win, b_in, We1[0, :_H], be1[0], We1[0, _H:2 * _H])
    out = None
    for l in range(_DEPTH):
        pg, qg, xs, xd = _sc_gather(p, q, x, src, dst)
        m, v = _tc_edge(pg, qg, xs, xd, We1[l, 2 * _H], We2[l], be2[l],
                        Wc1[l], bc1[l], Wc2[l], bc2[l])
        aggm, aggx = _sc_scatter(m, v, dst)
        if l < _DEPTH - 1:
            h, x, p, q = _tc_node(h, x, aggm, aggx, Wn1[l, :_H], Wn1[l, _H:],
                                  bn1[l], Wn2[l], bn2[l],
                                  We1[l + 1, :_H], be1[l + 1],
                                  We1[l + 1, _H:2 * _H])
        else:
            out, x = _tc_node_last(h, x, aggm, aggx, Wn1[l, :_H], Wn1[l, _H:],
                                   bn1[l], Wn2[l], bn2[l], Wout, b_out)
    return (out, x[:, :3])


# speedup vs baseline: 3.0911x; 3.0911x over previous
import functools

import jax
import jax.numpy as jnp
from jax import lax
from jax.experimental import pallas as pl
from jax.experimental.pallas import tpu as pltpu
from jax.experimental.pallas import tpu_sc as plsc

_N = 10000
_E = 320000
_H = 128
_DEPTH = 4
_MAX_IN_DEG = 10
_XP = 16
_C = 128
_NCHUNKS = _E // _C
_NC = 2
_NS = 16
_NW = _NC * _NS
_BE = 2000
_BN = 2000

_f32 = jnp.float32


def _silu(v):
    return v * (1.0 / (1.0 + jnp.exp(-v)))


def _sc_gather(p, q, xpad, src, dst):
    mesh = plsc.VectorSubcoreMesh(core_axis_name="c", subcore_axis_name="s",
                                  num_cores=_NC, num_subcores=_NS)
    out_type = (
        jax.ShapeDtypeStruct((_E, _H), _f32),
        jax.ShapeDtypeStruct((_E, _H), _f32),
        jax.ShapeDtypeStruct((_E, _XP), _f32),
        jax.ShapeDtypeStruct((_E, _XP), _f32),
    )
    scratch = [
        pltpu.VMEM((_C,), jnp.int32),
        pltpu.VMEM((_C,), jnp.int32),
        pltpu.VMEM((_C, _H), _f32),
        pltpu.VMEM((_C, _H), _f32),
        pltpu.VMEM((_C, _XP), _f32),
        pltpu.VMEM((_C, _XP), _f32),
        pltpu.SemaphoreType.DMA,
        pltpu.SemaphoreType.DMA,
        pltpu.SemaphoreType.DMA,
        pltpu.SemaphoreType.DMA,
    ]

    def body(p_h, q_h, x_h, src_h, dst_h, pg_h, qg_h, xs_h, xd_h,
             idx_s, idx_d, bufp, bufq, bufxs, bufxd, sp, sq, s1, s2):
        c = lax.axis_index("c")
        s = lax.axis_index("s")
        wid = s * _NC + c
        base_chunks = _NCHUNKS // _NW
        extra = _NCHUNKS - base_chunks * _NW
        nloc = base_chunks + jnp.where(wid < extra, 1, 0)

        def step(i, carry):
            base = (wid + _NW * i) * _C
            pltpu.sync_copy(dst_h.at[pl.ds(base, _C)], idx_d)
            pltpu.sync_copy(src_h.at[pl.ds(base, _C)], idx_s)
            cp = pltpu.async_copy(p_h.at[idx_d], bufp, sp)
            cq = pltpu.async_copy(q_h.at[idx_s], bufq, sq)
            cxs = pltpu.async_copy(x_h.at[idx_s], bufxs, s1)
            cxd = pltpu.async_copy(x_h.at[idx_d], bufxd, s2)
            cp.wait()
            cq.wait()
            cxs.wait()
            cxd.wait()
            pltpu.sync_copy(bufp, pg_h.at[pl.ds(base, _C)])
            pltpu.sync_copy(bufq, qg_h.at[pl.ds(base, _C)])
            pltpu.sync_copy(bufxs, xs_h.at[pl.ds(base, _C)])
            pltpu.sync_copy(bufxd, xd_h.at[pl.ds(base, _C)])
            return carry

        lax.fori_loop(0, nloc, step, 0)

    return pl.kernel(body, out_type=out_type, mesh=mesh, scratch_types=scratch,
                     compiler_params=pltpu.CompilerParams(use_tc_tiling_on_sc=False))(
        p, q, xpad, src, dst)


def _sc_scatter(m, v, dst):
    mesh = plsc.VectorSubcoreMesh(core_axis_name="c", subcore_axis_name="s",
                                  num_cores=_NC, num_subcores=_NS)
    out_type = (
        jax.ShapeDtypeStruct((_NC, _N, _H), _f32),
        jax.ShapeDtypeStruct((_NC, _N, _XP), _f32),
    )
    scratch = [
        pltpu.VMEM((_C,), jnp.int32),
        pltpu.VMEM((_C, _H), _f32),
        pltpu.VMEM((_C, _XP), _f32),
        pltpu.VMEM_SHARED((_N, _H), _f32),
        pltpu.VMEM_SHARED((_N, _XP), _f32),
    ]
    rpt = _N // _NS
    zc = 125

    def body(m_h, v_h, dst_h, aggm_h, aggx_h, idx_d, bufm, bufv, shm, shx):
        c = lax.axis_index("c")
        s = lax.axis_index("s")

        def zm(t, carry):
            r = t // (_H // 16)
            k = t % (_H // 16)
            bufm[r, pl.ds(k * 16, 16)] = jnp.zeros((16,), _f32)
            return carry

        lax.fori_loop(0, _C * (_H // 16), zm, 0)

        def zv(t, carry):
            bufv[t, :] = jnp.zeros((_XP,), _f32)
            return carry

        lax.fori_loop(0, _C, zv, 0)

        for r in range(rpt // zc):
            pltpu.sync_copy(bufm.at[pl.ds(0, zc)],
                            shm.at[pl.ds(s * rpt + r * zc, zc)])
            pltpu.sync_copy(bufv.at[pl.ds(0, zc)],
                            shx.at[pl.ds(s * rpt + r * zc, zc)])
        plsc.subcore_barrier()

        percore = _NCHUNKS // _NC
        base_t = percore // _NS
        extra = percore - base_t * _NS
        nloc = base_t + jnp.where(s < extra, 1, 0)

        def step(i, carry):
            base = (c + _NC * (s + _NS * i)) * _C
            pltpu.sync_copy(dst_h.at[pl.ds(base, _C)], idx_d)
            pltpu.sync_copy(m_h.at[pl.ds(base, _C)], bufm)
            pltpu.sync_copy(v_h.at[pl.ds(base, _C)], bufv)
            pltpu.sync_copy(bufm, shm.at[idx_d], add=True)
            pltpu.sync_copy(bufv, shx.at[idx_d], add=True)
            return carry

        lax.fori_loop(0, nloc, step, 0)
        plsc.subcore_barrier()

        pltpu.sync_copy(shm.at[pl.ds(s * rpt, rpt)],
                        aggm_h.at[c, pl.ds(s * rpt, rpt)])
        pltpu.sync_copy(shx.at[pl.ds(s * rpt, rpt)],
                        aggx_h.at[c, pl.ds(s * rpt, rpt)])

    return pl.kernel(body, out_type=out_type, mesh=mesh, scratch_types=scratch,
                     compiler_params=pltpu.CompilerParams(use_tc_tiling_on_sc=False))(
        m, v, dst)


def _full2(shape):
    return pl.BlockSpec(shape, lambda i: (0, 0))


def _tc_embed(feat, Win, b_in):
    def body(f_r, win_r, bin_r, h_r):
        h_r[...] = jnp.dot(f_r[...], win_r[...], preferred_element_type=_f32) + bin_r[...]

    row = pl.BlockSpec((_BN, _H), lambda i: (i, 0))
    return pl.pallas_call(
        body,
        grid=(_N // _BN,),
        in_specs=[row, _full2((_H, _H)), _full2((1, _H))],
        out_specs=row,
        out_shape=jax.ShapeDtypeStruct((_N, _H), _f32),
    )(feat, Win, b_in.reshape(1, _H))


def _tc_edge(hd, hs, xs, xd, We1l, be1l, We2l, be2l, Wc1l, bc1l, Wc2l, bc2l):
    def body(hd_r, hs_r, xs_r, xd_r, we1_r, be1_r, we2_r, be2_r, wc1_r, bc1_r,
             wc2_r, bc2_r, m_r, v_r):
        diff = xd_r[...] - xs_r[...]
        r2 = jnp.sum(diff * diff, axis=-1, keepdims=True)
        em = jnp.concatenate([hd_r[...], hs_r[...], r2], axis=-1)
        u = _silu(jnp.dot(em, we1_r[...], preferred_element_type=_f32) + be1_r[...])
        m = _silu(jnp.dot(u, we2_r[...], preferred_element_type=_f32) + be2_r[...])
        t = _silu(jnp.dot(m, wc1_r[...], preferred_element_type=_f32) + bc1_r[...])
        cw = jnp.dot(t, wc2_r[...], preferred_element_type=_f32) + bc2_r[...]
        m_r[...] = m
        v_r[...] = diff * cw

    erow = pl.BlockSpec((_BE, _H), lambda i: (i, 0))
    xrow = pl.BlockSpec((_BE, _XP), lambda i: (i, 0))
    return pl.pallas_call(
        body,
        grid=(_E // _BE,),
        in_specs=[erow, erow, xrow, xrow, _full2((2 * _H + 1, _H)),
                  _full2((1, _H)), _full2((_H, _H)), _full2((1, _H)),
                  _full2((_H, _H)), _full2((1, _H)),
                  _full2((_H, 1)), _full2((1, 1))],
        out_specs=[erow, xrow],
        out_shape=[jax.ShapeDtypeStruct((_E, _H), _f32),
                   jax.ShapeDtypeStruct((_E, _XP), _f32)],
    )(hd, hs, xs, xd, We1l, be1l.reshape(1, _H), We2l, be2l.reshape(1, _H),
      Wc1l, bc1l.reshape(1, _H), Wc2l, bc2l.reshape(1, 1))


def _tc_node(h, x, aggm, aggx, Wn1l, bn1l, Wn2l, bn2l):
    def body(h_r, x_r, am_r, ax_r, wn1_r, bn1_r, wn2_r, bn2_r, h2_r, x2_r):
        am = am_r[0] + am_r[1]
        ax = ax_r[0] + ax_r[1]
        nm = jnp.concatenate([h_r[...], am], axis=-1)
        g = _silu(jnp.dot(nm, wn1_r[...], preferred_element_type=_f32) + bn1_r[...])
        h2_r[...] = h_r[...] + jnp.dot(g, wn2_r[...], preferred_element_type=_f32) + bn2_r[...]
        x2_r[...] = x_r[...] + ax / _MAX_IN_DEG

    row = pl.BlockSpec((_BN, _H), lambda i: (i, 0))
    xrow = pl.BlockSpec((_BN, _XP), lambda i: (i, 0))
    amrow = pl.BlockSpec((_NC, _BN, _H), lambda i: (0, i, 0))
    axrow = pl.BlockSpec((_NC, _BN, _XP), lambda i: (0, i, 0))
    return pl.pallas_call(
        body,
        grid=(_N // _BN,),
        in_specs=[row, xrow, amrow, axrow, _full2((2 * _H, _H)),
                  _full2((1, _H)), _full2((_H, _H)), _full2((1, _H))],
        out_specs=[row, xrow],
        out_shape=[jax.ShapeDtypeStruct((_N, _H), _f32),
                   jax.ShapeDtypeStruct((_N, _XP), _f32)],
    )(h, x, aggm, aggx, Wn1l, bn1l.reshape(1, _H), Wn2l, bn2l.reshape(1, _H))


def _tc_node_last(h, x, aggm, aggx, Wn1l, bn1l, Wn2l, bn2l, Wout, b_out):
    def body(h_r, x_r, am_r, ax_r, wn1_r, bn1_r, wn2_r, bn2_r,
             wo_r, bo_r, o_r, x2_r):
        am = am_r[0] + am_r[1]
        ax = ax_r[0] + ax_r[1]
        nm = jnp.concatenate([h_r[...], am], axis=-1)
        g = _silu(jnp.dot(nm, wn1_r[...], preferred_element_type=_f32) + bn1_r[...])
        h2 = h_r[...] + jnp.dot(g, wn2_r[...], preferred_element_type=_f32) + bn2_r[...]
        o_r[...] = jnp.dot(h2, wo_r[...], preferred_element_type=_f32) + bo_r[...]
        x2_r[...] = x_r[...] + ax / _MAX_IN_DEG

    row = pl.BlockSpec((_BN, _H), lambda i: (i, 0))
    xrow = pl.BlockSpec((_BN, _XP), lambda i: (i, 0))
    amrow = pl.BlockSpec((_NC, _BN, _H), lambda i: (0, i, 0))
    axrow = pl.BlockSpec((_NC, _BN, _XP), lambda i: (0, i, 0))
    return pl.pallas_call(
        body,
        grid=(_N // _BN,),
        in_specs=[row, xrow, amrow, axrow, _full2((2 * _H, _H)),
                  _full2((1, _H)), _full2((_H, _H)), _full2((1, _H)),
                  _full2((_H, _H)), _full2((1, _H))],
        out_specs=[row, xrow],
        out_shape=[jax.ShapeDtypeStruct((_N, _H), _f32),
                   jax.ShapeDtypeStruct((_N, _XP), _f32)],
    )(h, x, aggm, aggx, Wn1l, bn1l.reshape(1, _H), Wn2l,
      bn2l.reshape(1, _H), Wout, b_out.reshape(1, _H))


def kernel(feat, coordinate, edge_index, Win, b_in, Wout, b_out,
           We1, be1, We2, be2, Wc1, bc1, Wc2, bc2, Wn1, bn1, Wn2, bn2):
    src = edge_index[0]
    dst = edge_index[1]
    x = jnp.pad(coordinate, ((0, 0), (0, _XP - 3)))

    h = _tc_embed(feat, Win, b_in)
    out = None
    for l in range(_DEPTH):
        hd, hs, xs, xd = _sc_gather(h, h, x, src, dst)
        m, v = _tc_edge(hd, hs, xs, xd, We1[l], be1[l], We2[l], be2[l],
                        Wc1[l], bc1[l], Wc2[l], bc2[l])
        aggm, aggx = _sc_scatter(m, v, dst)
        if l < _DEPTH - 1:
            h, x = _tc_node(h, x, aggm, aggx, Wn1[l], bn1[l], Wn2[l], bn2[l])
        else:
            out, x = _tc_node_last(h, x, aggm, aggx, Wn1[l], bn1[l],
                                   Wn2[l], bn2[l], Wout, b_out)
    return (out, x[:, :3])
